# Initial kernel scaffold; baseline (speedup 1.0000x reference)
import jax
import jax.numpy as jnp
from jax.experimental import pallas as pl


def kernel(x, feature_bank, image_bank, mask_bank, top_k):
    raise NotImplementedError("write your pallas kernel here")


# speedup vs baseline: 54.0573x; 54.0573x over previous
import functools

import jax
import jax.numpy as jnp
from jax import lax
from jax.experimental import pallas as pl
from jax.experimental.pallas import tpu as pltpu
from jax.experimental.pallas import tpu_sc as plsc

_TOPK = 5
_PAD = 8
_KTILE = 1024
_NEG = -1e30


def _topk_body(x_ref, bank_ref, vals_ref, idx_ref, *, k_actual, ktile, q, topk):
    t = pl.program_id(0)
    big = jnp.int32(2**30)

    x = x_ref[...]
    xn = x / jnp.maximum(jnp.sqrt(jnp.sum(x * x, axis=1, keepdims=True)), 1e-8)
    b = bank_ref[...]
    bn = b / jnp.maximum(jnp.sqrt(jnp.sum(b * b, axis=1, keepdims=True)), 1e-8)
    s = lax.dot_general(xn, bn, (((1,), (1,)), ((), ())),
                        preferred_element_type=jnp.float32)

    col1 = lax.broadcasted_iota(jnp.int32, (1, ktile), 1)
    s = jnp.where(col1 < (k_actual - t * ktile), s, _NEG)

    col = lax.broadcasted_iota(jnp.int32, (q, ktile), 1)
    tv, ti = [], []
    sw = s
    for _ in range(topk):
        m = jnp.max(sw, axis=1)
        a = jnp.min(jnp.where(sw == m[:, None], col, big), axis=1)
        tv.append(m)
        ti.append(a + t * ktile)
        sw = jnp.where(col == a[:, None], _NEG, sw)
    padv = [jnp.full((q,), _NEG, jnp.float32)] * (_PAD - topk)
    padi = [jnp.zeros((q,), jnp.int32)] * (_PAD - topk)
    tvals = jnp.stack(tv + padv, axis=1)
    tidx = jnp.stack(ti + padi, axis=1)

    first = t == 0
    rv = jnp.where(first, jnp.full((q, _PAD), _NEG, jnp.float32), vals_ref[...])
    ri = jnp.where(first, jnp.zeros((q, _PAD), jnp.int32), idx_ref[...])
    c = jnp.concatenate([rv, tvals], axis=1)
    ci = jnp.concatenate([ri, tidx], axis=1)
    lane = lax.broadcasted_iota(jnp.int32, (q, 2 * _PAD), 1)
    nv, ni = [], []
    for _ in range(topk):
        m = jnp.max(c, axis=1)
        a = jnp.min(jnp.where(c == m[:, None], lane, big), axis=1)
        nv.append(m)
        ni.append(jnp.sum(jnp.where(lane == a[:, None], ci, 0), axis=1))
        c = jnp.where(lane == a[:, None], _NEG, c)
    vals_ref[...] = jnp.stack(nv + padv, axis=1)
    idx_ref[...] = jnp.stack(ni + padi, axis=1)


def _topk_call(x, bank_padded, k_actual):
    q, d = x.shape
    kpad = bank_padded.shape[0]
    grid = (kpad // _KTILE,)
    body = functools.partial(_topk_body, k_actual=k_actual, ktile=_KTILE,
                             q=q, topk=_TOPK)
    return pl.pallas_call(
        body,
        grid=grid,
        in_specs=[
            pl.BlockSpec((q, d), lambda t: (0, 0)),
            pl.BlockSpec((_KTILE, d), lambda t: (t, 0)),
        ],
        out_specs=[
            pl.BlockSpec((q, _PAD), lambda t: (0, 0)),
            pl.BlockSpec((q, _PAD), lambda t: (0, 0)),
        ],
        out_shape=[
            jax.ShapeDtypeStruct((q, _PAD), jnp.float32),
            jax.ShapeDtypeStruct((q, _PAD), jnp.int32),
        ],
        compiler_params=pltpu.CompilerParams(
            dimension_semantics=("arbitrary",)),
    )(x, bank_padded)


def _gather_body(feat_hbm, img_hbm, mask_hbm, idx_hbm,
                 out_f, out_i, out_m,
                 idx_v, fv, iv, mv, semf, semi, semm,
                 *, per_worker, chunk, num_cores):
    wid = lax.axis_index("s") * num_cores + lax.axis_index("c")
    base = wid * per_worker
    for j in range(per_worker // chunk):
        off = base + j * chunk
        pltpu.sync_copy(idx_hbm.at[pl.ds(off, chunk)], idx_v)
        cf = pltpu.async_copy(feat_hbm.at[idx_v], fv, semf)
        ci = pltpu.async_copy(img_hbm.at[idx_v], iv, semi)
        cm = pltpu.async_copy(mask_hbm.at[idx_v], mv, semm)
        cf.wait()
        ci.wait()
        cm.wait()
        pltpu.sync_copy(fv, out_f.at[pl.ds(off, chunk)])
        pltpu.sync_copy(iv, out_i.at[pl.ds(off, chunk)])
        pltpu.sync_copy(mv, out_m.at[pl.ds(off, chunk)])


def _gather_call(feat, img, mask16, idx_flat):
    n = idx_flat.shape[0]
    d = feat.shape[1]
    dm = mask16.shape[1]
    info = plsc.get_sparse_core_info()
    nw = info.num_cores * info.num_subcores
    per_worker = n // nw
    chunk = per_worker
    while chunk > 128 or chunk % 8 != 0:
        chunk //= 2
    mesh = plsc.VectorSubcoreMesh(core_axis_name="c", subcore_axis_name="s")
    body = functools.partial(_gather_body, per_worker=per_worker, chunk=chunk,
                             num_cores=info.num_cores)
    gk = pl.kernel(
        body,
        mesh=mesh,
        out_type=[
            jax.ShapeDtypeStruct((n, d), jnp.float32),
            jax.ShapeDtypeStruct((n, d), jnp.float32),
            jax.ShapeDtypeStruct((n, dm), jnp.float32),
        ],
        scratch_types=[
            pltpu.VMEM((chunk,), jnp.int32),
            pltpu.VMEM((chunk, d), jnp.float32),
            pltpu.VMEM((chunk, d), jnp.float32),
            pltpu.VMEM((chunk, dm), jnp.float32),
            pltpu.SemaphoreType.DMA,
            pltpu.SemaphoreType.DMA,
            pltpu.SemaphoreType.DMA,
        ],
        compiler_params=pltpu.CompilerParams(use_tc_tiling_on_sc=False),
    )
    return gk(feat, img, mask16, idx_flat)


def _fuse_body(x_ref, gf_ref, r0_ref, valid_ref, out_ref, *, topk):
    v = valid_ref[0:1, :]
    r0 = r0_ref[0:1, :]
    rates = jnp.sum(r0 * v) / jnp.sum(v)
    e = jnp.exp(r0) * v
    w = rates * e / jnp.sum(e)
    acc = x_ref[...] * (1.0 - rates)
    for j in range(topk):
        acc = acc + gf_ref[j] * w[0:1, j:j + 1]
    out_ref[...] = acc


def _fuse_call(x, gf, r0, valid):
    q, d = x.shape
    return pl.pallas_call(
        functools.partial(_fuse_body, topk=_TOPK),
        out_shape=jax.ShapeDtypeStruct((q, d), jnp.float32),
    )(x, gf, r0, valid)


def kernel(x, feature_bank, image_bank, mask_bank, top_k):
    q, d = x.shape
    k = feature_bank.shape[0]
    mc = mask_bank.shape[1]

    kpad = -(-k // _KTILE) * _KTILE
    bank_p = jnp.pad(feature_bank, ((0, kpad - k), (0, 0)))
    vals, idx = _topk_call(x, bank_p, k)

    outall = idx[:, :_TOPK]
    idx_jm = outall.T.reshape(-1)

    mask16 = jnp.pad(mask_bank, ((0, 0), (0, 16 - mc)))
    gf_flat, gi_flat, gm_flat = _gather_call(
        feature_bank, image_bank, mask16, idx_jm)

    gf = gf_flat.reshape(_TOPK, q, d)
    valid = (jnp.arange(_PAD)[None, :] < jnp.minimum(top_k, _TOPK)
             ).astype(jnp.float32)
    r0 = lax.dynamic_slice(vals, (0, 0), (1, _PAD))
    out = _fuse_call(x, gf, r0, valid)

    gathered_feat = gf.transpose(1, 0, 2)
    gathered_img = gi_flat.reshape(_TOPK, q, d).transpose(1, 0, 2)
    gathered_mask = gm_flat.reshape(_TOPK, q, 16)[:, :, :mc].transpose(1, 0, 2)
    return out, gathered_feat, gathered_img, gathered_mask


# speedup vs baseline: 58.6586x; 1.0851x over previous
import functools

import jax
import jax.numpy as jnp
from jax import lax
from jax.experimental import pallas as pl
from jax.experimental.pallas import tpu as pltpu
from jax.experimental.pallas import tpu_sc as plsc

_TOPK = 5
_PAD = 8
_KTILE = 2048
_NEG = -1e30


def _extract_topk(s, col, topk, big):
    tv, ta = [], []
    pm = pa = None
    for i in range(topk):
        if i == 0:
            m = jnp.max(s, axis=1)
            a = jnp.min(jnp.where(s == m[:, None], col, big), axis=1)
        else:
            remain = (s < pm[:, None]) | ((s == pm[:, None])
                                          & (col > pa[:, None]))
            m = jnp.max(jnp.where(remain, s, _NEG), axis=1)
            a = jnp.min(jnp.where((s == m[:, None]) & remain, col, big),
                        axis=1)
        tv.append(m)
        ta.append(a)
        pm, pa = m, a
    return tv, ta


def _topk_tile_body(x_ref, bank_ref, vals_ref, idx_ref,
                    *, k_actual, ktile, q, topk):
    t = pl.program_id(0)
    big = jnp.int32(2**30)

    x = x_ref[...]
    xn = x / jnp.maximum(jnp.sqrt(jnp.sum(x * x, axis=1, keepdims=True)), 1e-8)
    b = bank_ref[...]
    bn = b / jnp.maximum(jnp.sqrt(jnp.sum(b * b, axis=1, keepdims=True)), 1e-8)
    s = lax.dot_general(xn, bn, (((1,), (1,)), ((), ())),
                        preferred_element_type=jnp.float32)

    col1 = lax.broadcasted_iota(jnp.int32, (1, ktile), 1)
    s = jnp.where(col1 < (k_actual - t * ktile), s, _NEG)

    col = lax.broadcasted_iota(jnp.int32, (q, ktile), 1)
    tv, ta = _extract_topk(s, col, topk, big)
    padv = [jnp.full((q,), _NEG, jnp.float32)] * (_PAD - topk)
    padi = [jnp.zeros((q,), jnp.int32)] * (_PAD - topk)
    vals_ref[0] = jnp.stack(tv + padv, axis=1)
    idx_ref[0] = jnp.stack([a + t * ktile for a in ta] + padi, axis=1)


def _topk_tile_call(x, bank, k_actual):
    q, d = x.shape
    nt = -(-k_actual // _KTILE)
    body = functools.partial(_topk_tile_body, k_actual=k_actual, ktile=_KTILE,
                             q=q, topk=_TOPK)
    return pl.pallas_call(
        body,
        grid=(nt,),
        in_specs=[
            pl.BlockSpec((q, d), lambda t: (0, 0)),
            pl.BlockSpec((_KTILE, d), lambda t: (t, 0)),
        ],
        out_specs=[
            pl.BlockSpec((1, q, _PAD), lambda t: (t, 0, 0)),
            pl.BlockSpec((1, q, _PAD), lambda t: (t, 0, 0)),
        ],
        out_shape=[
            jax.ShapeDtypeStruct((nt, q, _PAD), jnp.float32),
            jax.ShapeDtypeStruct((nt, q, _PAD), jnp.int32),
        ],
        compiler_params=pltpu.CompilerParams(
            dimension_semantics=("arbitrary",)),
    )(x, bank)


def _topk_merge_body(cv_ref, ci_ref, vals_ref, idx_ref, *, q, width, topk):
    big = jnp.int32(2**30)
    cv = cv_ref[...]
    ci = ci_ref[...]
    lane = lax.broadcasted_iota(jnp.int32, (q, width), 1)
    tv, ta = _extract_topk(cv, lane, topk, big)
    nv, ni = [], []
    for m, a in zip(tv, ta):
        nv.append(m)
        ni.append(jnp.sum(jnp.where(lane == a[:, None], ci, 0), axis=1))
    padv = [jnp.full((q,), _NEG, jnp.float32)] * (_PAD - topk)
    padi = [jnp.zeros((q,), jnp.int32)] * (_PAD - topk)
    vals_ref[...] = jnp.stack(nv + padv, axis=1)
    idx_ref[...] = jnp.stack(ni + padi, axis=1)


def _topk_merge_call(cand_vals, cand_idx):
    q, width = cand_vals.shape
    body = functools.partial(_topk_merge_body, q=q, width=width, topk=_TOPK)
    return pl.pallas_call(
        body,
        out_shape=[
            jax.ShapeDtypeStruct((q, _PAD), jnp.float32),
            jax.ShapeDtypeStruct((q, _PAD), jnp.int32),
        ],
    )(cand_vals, cand_idx)


def _topk_call(x, bank, k_actual):
    q = x.shape[0]
    tvals, tidx = _topk_tile_call(x, bank, k_actual)
    nt = tvals.shape[0]
    cv = tvals.transpose(1, 0, 2).reshape(q, nt * _PAD)
    ci = tidx.transpose(1, 0, 2).reshape(q, nt * _PAD)
    return _topk_merge_call(cv, ci)


def _gather_body(feat_hbm, img_hbm, mask_hbm, idx_hbm,
                 out_f, out_i, out_m,
                 idx_v, fv, iv, mv, semf, semi, semm,
                 *, per_worker, chunk, num_cores):
    wid = lax.axis_index("s") * num_cores + lax.axis_index("c")
    base = wid * per_worker
    for j in range(per_worker // chunk):
        off = base + j * chunk
        pltpu.sync_copy(idx_hbm.at[pl.ds(off, chunk)], idx_v)
        cf = pltpu.async_copy(feat_hbm.at[idx_v], fv, semf)
        ci = pltpu.async_copy(img_hbm.at[idx_v], iv, semi)
        cm = pltpu.async_copy(mask_hbm.at[idx_v], mv, semm)
        cf.wait()
        ci.wait()
        cm.wait()
        pltpu.sync_copy(fv, out_f.at[pl.ds(off, chunk)])
        pltpu.sync_copy(iv, out_i.at[pl.ds(off, chunk)])
        pltpu.sync_copy(mv, out_m.at[pl.ds(off, chunk)])


def _gather_call(feat, img, mask16, idx_flat):
    n = idx_flat.shape[0]
    d = feat.shape[1]
    dm = mask16.shape[1]
    info = plsc.get_sparse_core_info()
    nw = info.num_cores * info.num_subcores
    per_worker = n // nw
    chunk = per_worker
    while chunk > 128 or chunk % 8 != 0:
        chunk //= 2
    mesh = plsc.VectorSubcoreMesh(core_axis_name="c", subcore_axis_name="s")
    body = functools.partial(_gather_body, per_worker=per_worker, chunk=chunk,
                             num_cores=info.num_cores)
    gk = pl.kernel(
        body,
        mesh=mesh,
        out_type=[
            jax.ShapeDtypeStruct((n, d), jnp.float32),
            jax.ShapeDtypeStruct((n, d), jnp.float32),
            jax.ShapeDtypeStruct((n, dm), jnp.float32),
        ],
        scratch_types=[
            pltpu.VMEM((chunk,), jnp.int32),
            pltpu.VMEM((chunk, d), jnp.float32),
            pltpu.VMEM((chunk, d), jnp.float32),
            pltpu.VMEM((chunk, dm), jnp.float32),
            pltpu.SemaphoreType.DMA,
            pltpu.SemaphoreType.DMA,
            pltpu.SemaphoreType.DMA,
        ],
        compiler_params=pltpu.CompilerParams(use_tc_tiling_on_sc=False),
    )
    return gk(feat, img, mask16, idx_flat)


def _fuse_body(x_ref, gf_ref, r0_ref, valid_ref, out_ref, *, topk):
    v = valid_ref[0:1, :]
    r0 = r0_ref[0:1, :]
    rates = jnp.sum(r0 * v) / jnp.sum(v)
    e = jnp.exp(r0) * v
    w = rates * e / jnp.sum(e)
    acc = x_ref[...] * (1.0 - rates)
    for j in range(topk):
        acc = acc + gf_ref[j] * w[0:1, j:j + 1]
    out_ref[...] = acc


def _fuse_call(x, gf, r0, valid):
    q, d = x.shape
    return pl.pallas_call(
        functools.partial(_fuse_body, topk=_TOPK),
        out_shape=jax.ShapeDtypeStruct((q, d), jnp.float32),
    )(x, gf, r0, valid)


def kernel(x, feature_bank, image_bank, mask_bank, top_k):
    q, d = x.shape
    k = feature_bank.shape[0]
    mc = mask_bank.shape[1]

    vals, idx = _topk_call(x, feature_bank, k)

    outall = idx[:, :_TOPK]
    idx_jm = outall.T.reshape(-1)

    mask16 = jnp.pad(mask_bank, ((0, 0), (0, 16 - mc)))
    gf_flat, gi_flat, gm_flat = _gather_call(
        feature_bank, image_bank, mask16, idx_jm)

    gf = gf_flat.reshape(_TOPK, q, d)
    valid = (jnp.arange(_PAD)[None, :] < jnp.minimum(top_k, _TOPK)
             ).astype(jnp.float32)
    r0 = lax.dynamic_slice(vals, (0, 0), (1, _PAD))
    out = _fuse_call(x, gf, r0, valid)

    gathered_feat = gf.transpose(1, 0, 2)
    gathered_img = gi_flat.reshape(_TOPK, q, d).transpose(1, 0, 2)
    gathered_mask = gm_flat.reshape(_TOPK, q, 16)[:, :, :mc].transpose(1, 0, 2)
    return out, gathered_feat, gathered_img, gathered_mask


# speedup vs baseline: 62.0891x; 1.0585x over previous
import functools

import jax
import jax.numpy as jnp
from jax import lax
from jax.experimental import pallas as pl
from jax.experimental.pallas import tpu as pltpu
from jax.experimental.pallas import tpu_sc as plsc

_TOPK = 5
_PAD = 8
_KTILE = 2048
_NEG = -1e30


def _extract_topk(s, col, topk):
    big = 3e38
    tv, ta = [], []
    pm = pa = None
    for i in range(topk):
        if i == 0:
            m = jnp.max(s, axis=1)
            a = jnp.min(jnp.where(s == m[:, None], col, big), axis=1)
        else:
            remain = (s < pm[:, None]) | ((s == pm[:, None])
                                          & (col > pa[:, None]))
            m = jnp.max(jnp.where(remain, s, _NEG), axis=1)
            a = jnp.min(jnp.where((s == m[:, None]) & remain, col, big),
                        axis=1)
        tv.append(m)
        ta.append(a)
        pm, pa = m, a
    return tv, ta


def _topk_tile_body(x_ref, bank_ref, vals_ref, idx_ref,
                    *, k_actual, ktile, q, topk):
    t = pl.program_id(0)

    x = x_ref[...]
    xn = x / jnp.maximum(jnp.sqrt(jnp.sum(x * x, axis=1, keepdims=True)), 1e-8)
    b = bank_ref[...]
    bn = b / jnp.maximum(jnp.sqrt(jnp.sum(b * b, axis=1, keepdims=True)), 1e-8)
    s = lax.dot_general(xn, bn, (((1,), (1,)), ((), ())),
                        preferred_element_type=jnp.float32)

    col1 = lax.broadcasted_iota(jnp.int32, (1, ktile), 1)
    s = jnp.where(col1 < (k_actual - t * ktile), s, _NEG)

    col = lax.broadcasted_iota(jnp.int32, (q, ktile), 1).astype(jnp.float32)
    tv, ta = _extract_topk(s, col, topk)
    padv = [jnp.full((q,), _NEG, jnp.float32)] * (_PAD - topk)
    base = (t * ktile).astype(jnp.float32)
    vals_ref[0] = jnp.stack(tv + padv, axis=1)
    padz = [jnp.zeros((q,), jnp.float32)] * (_PAD - topk)
    idx_ref[0] = jnp.stack([a + base for a in ta]
                           + padz, axis=1).astype(jnp.int32)


def _topk_tile_call(x, bank, k_actual):
    q, d = x.shape
    nt = -(-k_actual // _KTILE)
    body = functools.partial(_topk_tile_body, k_actual=k_actual, ktile=_KTILE,
                             q=q, topk=_TOPK)
    return pl.pallas_call(
        body,
        grid=(nt,),
        in_specs=[
            pl.BlockSpec((q, d), lambda t: (0, 0)),
            pl.BlockSpec((_KTILE, d), lambda t: (t, 0)),
        ],
        out_specs=[
            pl.BlockSpec((1, q, _PAD), lambda t: (t, 0, 0)),
            pl.BlockSpec((1, q, _PAD), lambda t: (t, 0, 0)),
        ],
        out_shape=[
            jax.ShapeDtypeStruct((nt, q, _PAD), jnp.float32),
            jax.ShapeDtypeStruct((nt, q, _PAD), jnp.int32),
        ],
        compiler_params=pltpu.CompilerParams(
            dimension_semantics=("arbitrary",)),
    )(x, bank)


def _topk_merge_body(cv_ref, ci_ref, vals_ref, idx_ref, *, q, width, topk):
    cv = cv_ref[...]
    ci = ci_ref[...]
    lane = lax.broadcasted_iota(jnp.int32, (q, width), 1).astype(jnp.float32)
    tv, ta = _extract_topk(cv, lane, topk)
    nv, ni = [], []
    for m, a in zip(tv, ta):
        nv.append(m)
        ni.append(jnp.sum(jnp.where(lane == a[:, None], ci, 0), axis=1))
    padv = [jnp.full((q,), _NEG, jnp.float32)] * (_PAD - topk)
    padi = [jnp.zeros((q,), jnp.int32)] * (_PAD - topk)
    vals_ref[...] = jnp.stack(nv + padv, axis=1)
    idx_ref[...] = jnp.stack(ni + padi, axis=1)


def _topk_merge_call(cand_vals, cand_idx):
    q, width = cand_vals.shape
    body = functools.partial(_topk_merge_body, q=q, width=width, topk=_TOPK)
    return pl.pallas_call(
        body,
        out_shape=[
            jax.ShapeDtypeStruct((q, _PAD), jnp.float32),
            jax.ShapeDtypeStruct((q, _PAD), jnp.int32),
        ],
    )(cand_vals, cand_idx)


def _topk_call(x, bank, k_actual):
    q = x.shape[0]
    tvals, tidx = _topk_tile_call(x, bank, k_actual)
    nt = tvals.shape[0]
    cv = tvals.transpose(1, 0, 2).reshape(q, nt * _PAD)
    ci = tidx.transpose(1, 0, 2).reshape(q, nt * _PAD)
    return _topk_merge_call(cv, ci)


def _gather_body(feat_hbm, img_hbm, mask_hbm, idx_hbm,
                 out_f, out_i, out_m,
                 idx_v, fv, iv, mv, semf, semi, semm,
                 *, per_worker, chunk, num_cores):
    wid = lax.axis_index("s") * num_cores + lax.axis_index("c")
    base = wid * per_worker
    for j in range(per_worker // chunk):
        off = base + j * chunk
        pltpu.sync_copy(idx_hbm.at[pl.ds(off, chunk)], idx_v)
        cf = pltpu.async_copy(feat_hbm.at[idx_v], fv, semf)
        ci = pltpu.async_copy(img_hbm.at[idx_v], iv, semi)
        cm = pltpu.async_copy(mask_hbm.at[idx_v], mv, semm)
        cf.wait()
        ci.wait()
        cm.wait()
        pltpu.sync_copy(fv, out_f.at[pl.ds(off, chunk)])
        pltpu.sync_copy(iv, out_i.at[pl.ds(off, chunk)])
        pltpu.sync_copy(mv, out_m.at[pl.ds(off, chunk)])


def _gather_call(feat, img, mask16, idx_flat):
    n = idx_flat.shape[0]
    d = feat.shape[1]
    dm = mask16.shape[1]
    info = plsc.get_sparse_core_info()
    nw = info.num_cores * info.num_subcores
    per_worker = n // nw
    chunk = per_worker
    while chunk > 128 or chunk % 8 != 0:
        chunk //= 2
    mesh = plsc.VectorSubcoreMesh(core_axis_name="c", subcore_axis_name="s")
    body = functools.partial(_gather_body, per_worker=per_worker, chunk=chunk,
                             num_cores=info.num_cores)
    gk = pl.kernel(
        body,
        mesh=mesh,
        out_type=[
            jax.ShapeDtypeStruct((n, d), jnp.float32),
            jax.ShapeDtypeStruct((n, d), jnp.float32),
            jax.ShapeDtypeStruct((n, dm), jnp.float32),
        ],
        scratch_types=[
            pltpu.VMEM((chunk,), jnp.int32),
            pltpu.VMEM((chunk, d), jnp.float32),
            pltpu.VMEM((chunk, d), jnp.float32),
            pltpu.VMEM((chunk, dm), jnp.float32),
            pltpu.SemaphoreType.DMA,
            pltpu.SemaphoreType.DMA,
            pltpu.SemaphoreType.DMA,
        ],
        compiler_params=pltpu.CompilerParams(use_tc_tiling_on_sc=False),
    )
    return gk(feat, img, mask16, idx_flat)


def _fuse_body(x_ref, gf_ref, r0_ref, valid_ref, out_ref, *, topk, d):
    v = valid_ref[0:1, :]
    r0 = r0_ref[0:1, :]
    rates = jnp.sum(r0 * v) / jnp.sum(v)
    e = jnp.exp(r0) * v
    w = rates * e / jnp.sum(e)
    acc = x_ref[...] * (1.0 - rates)
    for j in range(topk):
        acc = acc + gf_ref[:, j * d:(j + 1) * d] * w[0:1, j:j + 1]
    out_ref[...] = acc


def _fuse_call(x, gf2, r0, valid):
    q, d = x.shape
    return pl.pallas_call(
        functools.partial(_fuse_body, topk=_TOPK, d=d),
        out_shape=jax.ShapeDtypeStruct((q, d), jnp.float32),
    )(x, gf2, r0, valid)


def kernel(x, feature_bank, image_bank, mask_bank, top_k):
    q, d = x.shape
    k = feature_bank.shape[0]
    mc = mask_bank.shape[1]

    vals, idx = _topk_call(x, feature_bank, k)

    outall = idx[:, :_TOPK]
    idx_flat = outall.reshape(-1)

    mask16 = jnp.pad(mask_bank, ((0, 0), (0, 16 - mc)))
    gf_flat, gi_flat, gm_flat = _gather_call(
        feature_bank, image_bank, mask16, idx_flat)

    valid = (jnp.arange(_PAD)[None, :] < jnp.minimum(top_k, _TOPK)
             ).astype(jnp.float32)
    r0 = lax.dynamic_slice(vals, (0, 0), (1, _PAD))
    out = _fuse_call(x, gf_flat.reshape(q, _TOPK * d), r0, valid)

    gathered_feat = gf_flat.reshape(q, _TOPK, d)
    gathered_img = gi_flat.reshape(q, _TOPK, d)
    gathered_mask = gm_flat.reshape(q, _TOPK, 16)[:, :, :mc]
    return out, gathered_feat, gathered_img, gathered_mask
